# baseline (speedup 1.0000x reference)
import functools

import jax
import jax.numpy as jnp
from jax.experimental import pallas as pl
from jax.experimental.pallas import tpu as pltpu


@functools.partial(jax.jit, static_argnums=(0, 1, 2))
def _pos_embed_tc(B, H, W, row_embed, col_embed):
    D = row_embed.shape[1]

    NS = 8
    HH = H // NS

    def body(row_ref, col_ref, o_hbm, scratch, sems):
        col = col_ref[:W, :]
        copies = []
        for h in range(NS):
            sl = pl.ds(h * HH, HH)
            row = row_ref[sl, :]
            scratch[sl, :, :D] = jnp.broadcast_to(col[None, :, :],
                                                  (HH, W, D))
            scratch[sl, :, D:] = jnp.broadcast_to(row[:, None, :],
                                                  (HH, W, D))
            for b in range(B):
                c = pltpu.make_async_copy(
                    scratch.at[sl], o_hbm.at[b, sl], sems.at[h, b])
                c.start()
                copies.append(c)
        for c in copies:
            c.wait()

    out = pl.pallas_call(
        body,
        in_specs=[
            pl.BlockSpec(row_embed.shape, lambda: (0, 0)),
            pl.BlockSpec(col_embed.shape, lambda: (0, 0)),
        ],
        out_specs=pl.BlockSpec(memory_space=pl.ANY),
        out_shape=jax.ShapeDtypeStruct((B, H, W, 2 * D), jnp.float32),
        scratch_shapes=[
            pltpu.VMEM((H, W, 2 * D), jnp.float32),
            pltpu.SemaphoreType.DMA((NS, B)),
        ],
    )(row_embed, col_embed)
    return jnp.transpose(out, (0, 3, 1, 2))


def kernel(x, row_embed, col_embed):
    B = x.shape[0]
    H, W = x.shape[-2], x.shape[-1]
    return _pos_embed_tc(B, H, W, row_embed, col_embed)


# speedup vs baseline: 1.0244x; 1.0244x over previous
import functools

import jax
import jax.numpy as jnp
from jax.experimental import pallas as pl
from jax.experimental.pallas import tpu as pltpu


@functools.partial(jax.jit, static_argnums=(0, 1, 2))
def _pos_embed_tc(B, H, W, row_embed, col_embed):
    D = row_embed.shape[1]

    NS = 4
    HH = H // NS

    def body(row_ref, col_ref, o_hbm, scratch, sems):
        col = col_ref[:W, :]
        copies = []
        for h in range(NS):
            sl = pl.ds(h * HH, HH)
            row = row_ref[sl, :]
            scratch[sl, :, :D] = jnp.broadcast_to(col[None, :, :],
                                                  (HH, W, D))
            scratch[sl, :, D:] = jnp.broadcast_to(row[:, None, :],
                                                  (HH, W, D))
            for b in range(B):
                c = pltpu.make_async_copy(
                    scratch.at[sl], o_hbm.at[b, sl], sems.at[h, b])
                c.start()
                copies.append(c)
        for c in copies:
            c.wait()

    out = pl.pallas_call(
        body,
        in_specs=[
            pl.BlockSpec(row_embed.shape, lambda: (0, 0)),
            pl.BlockSpec(col_embed.shape, lambda: (0, 0)),
        ],
        out_specs=pl.BlockSpec(memory_space=pl.ANY),
        out_shape=jax.ShapeDtypeStruct((B, H, W, 2 * D), jnp.float32),
        scratch_shapes=[
            pltpu.VMEM((H, W, 2 * D), jnp.float32),
            pltpu.SemaphoreType.DMA((NS, B)),
        ],
    )(row_embed, col_embed)
    return jnp.transpose(out, (0, 3, 1, 2))


def kernel(x, row_embed, col_embed):
    B = x.shape[0]
    H, W = x.shape[-2], x.shape[-1]
    return _pos_embed_tc(B, H, W, row_embed, col_embed)


# speedup vs baseline: 1.0249x; 1.0004x over previous
import functools

import jax
import jax.numpy as jnp
from jax.experimental import pallas as pl
from jax.experimental.pallas import tpu as pltpu


@functools.partial(jax.jit, static_argnums=(0, 1, 2))
def _pos_embed_tc(B, H, W, row_embed, col_embed):
    D = row_embed.shape[1]

    splits = [H // 8, H // 8, H // 4, H // 2]
    starts = [sum(splits[:k]) for k in range(len(splits))]
    NS = len(splits)

    def body(row_ref, col_ref, o_hbm, scratch, sems):
        col = col_ref[:W, :]
        copies = []
        for h in range(NS):
            HH = splits[h]
            sl = pl.ds(starts[h], HH)
            row = row_ref[sl, :]
            scratch[sl, :, :D] = jnp.broadcast_to(col[None, :, :],
                                                  (HH, W, D))
            scratch[sl, :, D:] = jnp.broadcast_to(row[:, None, :],
                                                  (HH, W, D))
            for b in range(B):
                c = pltpu.make_async_copy(
                    scratch.at[sl], o_hbm.at[b, sl], sems.at[h, b])
                c.start()
                copies.append(c)
        for c in copies:
            c.wait()

    out = pl.pallas_call(
        body,
        in_specs=[
            pl.BlockSpec(row_embed.shape, lambda: (0, 0)),
            pl.BlockSpec(col_embed.shape, lambda: (0, 0)),
        ],
        out_specs=pl.BlockSpec(memory_space=pl.ANY),
        out_shape=jax.ShapeDtypeStruct((B, H, W, 2 * D), jnp.float32),
        scratch_shapes=[
            pltpu.VMEM((H, W, 2 * D), jnp.float32),
            pltpu.SemaphoreType.DMA((NS, B)),
        ],
    )(row_embed, col_embed)
    return jnp.transpose(out, (0, 3, 1, 2))


def kernel(x, row_embed, col_embed):
    B = x.shape[0]
    H, W = x.shape[-2], x.shape[-1]
    return _pos_embed_tc(B, H, W, row_embed, col_embed)
